# baseline (speedup 1.0000x reference)
import functools
import math
import random

import numpy as np
import jax
import jax.numpy as jnp
from jax import lax
from jax.experimental import pallas as pl
from jax.experimental.pallas import tpu as pltpu
from jax.experimental.pallas import tpu_sc as plsc

LANES = 16


def _distortion_indices(h, w, a_max=3, r_max=0.7):
    random.seed(0)
    cols = h
    rows = w
    center_rows = int(np.round(random.uniform(1, rows - 2)))
    center_cols = int(np.round(random.uniform(1, cols - 2)))
    radius = random.uniform(0.03 * max(rows, cols), r_max * max(rows, cols))
    choice = random.randint(0, 1)
    spect_ratio1 = 1
    spect_ratio2 = 1
    if choice == 1:
        spect_ratio1 = random.uniform(1, a_max)
    else:
        spect_ratio2 = random.uniform(1, a_max)
    cols_np = np.arange(cols)
    rows_np = np.arange(rows)
    cols_np_t = np.tile(cols_np, (rows, 1))
    cols_pow = np.power(cols_np_t - center_cols, 2)
    rows_np_t = np.tile(rows_np, (cols, 1))
    rows_pow = np.power(rows_np_t - center_rows, 2)
    dis = np.sqrt(cols_pow + rows_pow.transpose())
    judge = (spect_ratio1 * np.abs(rows_np_t - center_rows).transpose()
             + spect_ratio2 * np.abs(cols_np_t - center_cols))
    index = np.where(judge <= radius)
    index_rows = np.rint(index[0]).astype(np.int64)
    index_cols = np.rint(index[1]).astype(np.int64)
    dis_val = dis[index]
    old_i = np.floor(dis_val / radius * (index_rows - center_rows) + center_rows)
    old_j = np.floor(dis_val / radius * (index_cols - center_cols) + center_cols)
    return (index_rows, index_cols,
            old_i.astype(np.int64), old_j.astype(np.int64))


def _band_patch_indices(h, w):
    ir, ic, oi, oj = _distortion_indices(h, w)
    oi = np.where(oi < 0, oi + h, oi).clip(0, h - 1)
    oj = np.where(oj < 0, oj + w, oj).clip(0, w - 1)
    r0 = int(min(ir.min(), oi.min())) // 8 * 8
    r1 = -(-(int(max(ir.max(), oi.max())) + 1) // 8) * 8
    nr = r1 - r0
    k = ir.shape[0]
    k_pad = math.ceil(k / LANES) * LANES
    pad = lambda a: np.concatenate(
        [a, np.full(k_pad - k, a[-1])]).astype(np.int32)
    return r0, nr, pad(oi - r0), pad(oj), pad(ir - r0), pad(ic)


@functools.cache
def _build_sc_call(n_slices, h, w, r0, nr, k_pad):
    info = plsc.get_sparse_core_info()
    nc, ns = info.num_cores, info.num_subcores
    n_workers = nc * ns
    assert n_slices % n_workers == 0
    per_worker = n_slices // n_workers
    n_chunks = k_pad // LANES
    mesh = plsc.VectorSubcoreMesh(core_axis_name="c", subcore_axis_name="s")

    assert per_worker % 2 == 0
    n_hi = h - r0 - nr

    @functools.partial(
        pl.kernel,
        mesh=mesh,
        out_type=jax.ShapeDtypeStruct((n_slices, h, w), jnp.float32),
        compiler_params=pltpu.CompilerParams(needs_layout_passes=False),
        scratch_types=[
            pltpu.VMEM((k_pad,), jnp.int32),
            pltpu.VMEM((k_pad,), jnp.int32),
            pltpu.VMEM((k_pad,), jnp.int32),
            pltpu.VMEM((k_pad,), jnp.int32),
            pltpu.VMEM((k_pad,), jnp.float32),
            pltpu.VMEM((nr, w), jnp.float32),
            pltpu.VMEM((nr, w), jnp.float32),
            pltpu.SemaphoreType.DMA,
            pltpu.SemaphoreType.DMA,
            pltpu.SemaphoreType.DMA,
            pltpu.SemaphoreType.DMA,
            pltpu.SemaphoreType.DMA,
        ],
    )
    def sc_patch(feat_hbm, gr_hbm, gc_hbm, dr_hbm, dc_hbm, out_hbm,
                 gr_v, gc_v, dr_v, dc_v, vals_v, band0_v, band1_v,
                 lsem0, lsem1, ssem0, ssem1, bulk_sem):
        wid = lax.axis_index("s") * nc + lax.axis_index("c")
        sl0 = wid * per_worker
        bufs = (band0_v, band1_v)
        lsems = (lsem0, lsem1)
        ssems = (ssem0, ssem1)
        bulk_lo = pltpu.async_copy(
            feat_hbm.at[pl.ds(sl0, per_worker), pl.ds(0, r0)],
            out_hbm.at[pl.ds(sl0, per_worker), pl.ds(0, r0)],
            bulk_sem)
        bulk_hi = pltpu.async_copy(
            feat_hbm.at[pl.ds(sl0, per_worker), pl.ds(r0 + nr, n_hi)],
            out_hbm.at[pl.ds(sl0, per_worker), pl.ds(r0 + nr, n_hi)],
            bulk_sem)
        pltpu.sync_copy(gr_hbm, gr_v)
        pltpu.sync_copy(gc_hbm, gc_v)
        pltpu.sync_copy(dr_hbm, dr_v)
        pltpu.sync_copy(dc_hbm, dc_v)

        def start_load(sl, b):
            pltpu.async_copy(feat_hbm.at[sl, pl.ds(r0, nr)], bufs[b],
                             lsems[b])

        def wait_load(b):
            pltpu.make_async_copy(feat_hbm.at[sl0, pl.ds(r0, nr)], bufs[b],
                                  lsems[b]).wait()

        def start_store(sl, b):
            pltpu.async_copy(bufs[b], out_hbm.at[sl, pl.ds(r0, nr)],
                             ssems[b])

        def wait_store(b):
            pltpu.make_async_copy(bufs[b], out_hbm.at[sl0, pl.ds(r0, nr)],
                                  ssems[b]).wait()

        start_load(sl0, 0)
        start_load(sl0 + 1, 1)

        def do_pair(g, _):
            for b in range(2):
                i = g * 2 + b
                buf = bufs[b]
                wait_load(b)

                def gather_chunk(t, _):
                    sel = pl.ds(t * LANES, LANES)
                    vals_v[sel] = plsc.load_gather(
                        buf, [gr_v[sel], gc_v[sel]])
                    return 0

                lax.fori_loop(0, n_chunks, gather_chunk, 0, unroll=8)

                def scatter_chunk(t, _):
                    sel = pl.ds(t * LANES, LANES)
                    plsc.store_scatter(buf, [dr_v[sel], dc_v[sel]],
                                       vals_v[sel])
                    return 0

                lax.fori_loop(0, n_chunks, scatter_chunk, 0, unroll=8)
                start_store(sl0 + i, b)

                @pl.when(i + 2 < per_worker)
                def _prefetch():
                    wait_store(b)
                    start_load(sl0 + i + 2, b)
            return 0

        lax.fori_loop(0, per_worker // 2, do_pair, 0)
        wait_store(0)
        wait_store(1)
        bulk_lo.wait()
        bulk_hi.wait()

    return sc_patch


def kernel(feature):
    b, c, h, w = feature.shape
    r0, nr, gr, gc, dr, dc = _band_patch_indices(h, w)
    n_slices = b * c
    sc_patch = _build_sc_call(n_slices, h, w, r0, nr, gr.shape[0])
    out = sc_patch(feature.reshape(n_slices, h, w),
                   jnp.asarray(gr), jnp.asarray(gc),
                   jnp.asarray(dr), jnp.asarray(dc))
    return out.reshape(b, c, h, w)


# speedup vs baseline: 6.0039x; 6.0039x over previous
import functools
import math
import random

import numpy as np
import jax
import jax.numpy as jnp
from jax import lax
from jax.experimental import pallas as pl
from jax.experimental.pallas import tpu as pltpu
from jax.experimental.pallas import tpu_sc as plsc

LANES = 16


def _distortion_indices(h, w, a_max=3, r_max=0.7):
    random.seed(0)
    cols = h
    rows = w
    center_rows = int(np.round(random.uniform(1, rows - 2)))
    center_cols = int(np.round(random.uniform(1, cols - 2)))
    radius = random.uniform(0.03 * max(rows, cols), r_max * max(rows, cols))
    choice = random.randint(0, 1)
    spect_ratio1 = 1
    spect_ratio2 = 1
    if choice == 1:
        spect_ratio1 = random.uniform(1, a_max)
    else:
        spect_ratio2 = random.uniform(1, a_max)
    cols_np = np.arange(cols)
    rows_np = np.arange(rows)
    cols_np_t = np.tile(cols_np, (rows, 1))
    cols_pow = np.power(cols_np_t - center_cols, 2)
    rows_np_t = np.tile(rows_np, (cols, 1))
    rows_pow = np.power(rows_np_t - center_rows, 2)
    dis = np.sqrt(cols_pow + rows_pow.transpose())
    judge = (spect_ratio1 * np.abs(rows_np_t - center_rows).transpose()
             + spect_ratio2 * np.abs(cols_np_t - center_cols))
    index = np.where(judge <= radius)
    index_rows = np.rint(index[0]).astype(np.int64)
    index_cols = np.rint(index[1]).astype(np.int64)
    dis_val = dis[index]
    old_i = np.floor(dis_val / radius * (index_rows - center_rows) + center_rows)
    old_j = np.floor(dis_val / radius * (index_cols - center_cols) + center_cols)
    return (index_rows, index_cols,
            old_i.astype(np.int64), old_j.astype(np.int64))


def _flat_patch_indices(h, w):
    ir, ic, oi, oj = _distortion_indices(h, w)
    oi = np.where(oi < 0, oi + h, oi).clip(0, h - 1)
    oj = np.where(oj < 0, oj + w, oj).clip(0, w - 1)
    src = (oi * w + oj).astype(np.int32)
    dst = (ir * w + ic).astype(np.int32)
    k = src.shape[0]
    k_pad = math.ceil(k / LANES) * LANES
    src = np.concatenate([src, np.full(k_pad - k, src[-1], np.int32)])
    dst = np.concatenate([dst, np.full(k_pad - k, dst[-1], np.int32)])
    return src, dst


@functools.cache
def _build_sc_call(n_slices, hw, k_pad):
    info = plsc.get_sparse_core_info()
    nc, ns = info.num_cores, info.num_subcores
    n_workers = nc * ns
    assert n_slices % n_workers == 0
    per_worker = n_slices // n_workers
    assert per_worker % 2 == 0
    n_chunks = k_pad // LANES
    mesh = plsc.VectorSubcoreMesh(core_axis_name="c", subcore_axis_name="s")

    @functools.partial(
        pl.kernel,
        mesh=mesh,
        out_type=jax.ShapeDtypeStruct((n_slices, hw), jnp.float32),
        compiler_params=pltpu.CompilerParams(needs_layout_passes=False),
        scratch_types=[
            pltpu.VMEM((k_pad,), jnp.int32),
            pltpu.VMEM((k_pad,), jnp.int32),
            pltpu.VMEM((k_pad,), jnp.float32),
            pltpu.VMEM((hw,), jnp.float32),
            pltpu.VMEM((hw,), jnp.float32),
            pltpu.SemaphoreType.DMA,
            pltpu.SemaphoreType.DMA,
            pltpu.SemaphoreType.DMA,
            pltpu.SemaphoreType.DMA,
        ],
    )
    def sc_patch(feat_hbm, src_hbm, dst_hbm, out_hbm,
                 src_v, dst_v, vals_v, slice0_v, slice1_v,
                 lsem0, lsem1, ssem0, ssem1):
        wid = lax.axis_index("s") * nc + lax.axis_index("c")
        sl0 = wid * per_worker
        bufs = (slice0_v, slice1_v)
        lsems = (lsem0, lsem1)
        ssems = (ssem0, ssem1)
        pltpu.sync_copy(src_hbm, src_v)
        pltpu.sync_copy(dst_hbm, dst_v)

        def start_load(sl, b):
            pltpu.async_copy(feat_hbm.at[sl], bufs[b], lsems[b])

        def wait_load(b):
            pltpu.make_async_copy(feat_hbm.at[sl0], bufs[b], lsems[b]).wait()

        def start_store(sl, b):
            pltpu.async_copy(bufs[b], out_hbm.at[sl], ssems[b])

        def wait_store(b):
            pltpu.make_async_copy(bufs[b], out_hbm.at[sl0], ssems[b]).wait()

        start_load(sl0, 0)
        start_load(sl0 + 1, 1)

        def do_pair(g, _):
            for b in range(2):
                i = g * 2 + b
                buf = bufs[b]
                wait_load(b)

                def gather_chunk(t, _):
                    sel = pl.ds(t * LANES, LANES)
                    vals_v[sel] = plsc.load_gather(buf, [src_v[sel]])
                    return 0

                lax.fori_loop(0, n_chunks, gather_chunk, 0, unroll=8)

                def scatter_chunk(t, _):
                    sel = pl.ds(t * LANES, LANES)
                    plsc.store_scatter(buf, [dst_v[sel]], vals_v[sel])
                    return 0

                lax.fori_loop(0, n_chunks, scatter_chunk, 0, unroll=8)
                start_store(sl0 + i, b)

                @pl.when(i + 2 < per_worker)
                def _prefetch():
                    wait_store(b)
                    start_load(sl0 + i + 2, b)
            return 0

        lax.fori_loop(0, per_worker // 2, do_pair, 0)
        wait_store(0)
        wait_store(1)

    return sc_patch


def kernel(feature):
    b, c, h, w = feature.shape
    src, dst = _flat_patch_indices(h, w)
    n_slices, hw = b * c, h * w
    sc_patch = _build_sc_call(n_slices, hw, src.shape[0])
    out = sc_patch(feature.reshape(n_slices, hw),
                   jnp.asarray(src), jnp.asarray(dst))
    return out.reshape(b, c, h, w)
